# baseline (speedup 1.0000x reference)
import jax
import jax.numpy as jnp
from jax import lax
from jax.experimental import pallas as pl
from jax.experimental.pallas import tpu as pltpu
from jax.experimental.pallas import tpu_sc as plsc

_BATCH = 4096
_HIST = 50
_DIM = 64
_NC = 2
_NS = 16
_NW = _NC * _NS
_BPW = _BATCH // _NW
_CB = 16
_ITERS = _BPW // _CB
_HB = _CB // 2
_CHUNKS = [(0, range(16)), (16, range(16)), (32, range(16)),
           (34, range(14, 16))]


def _sc_body(x_hbm, tbl_hbm, out_hbm, xv, tl, buf, sem):
    wid = lax.axis_index("s") * _NC + lax.axis_index("c")
    pltpu.sync_copy(tbl_hbm, tl)
    q0 = [tl[0, pl.ds(c * 16, 16)] for c in range(4)]
    q1 = [tl[1, pl.ds(c * 16, 16)] for c in range(4)]

    def out_copy(bb, b, sem_slot):
        return pltpu.make_async_copy(
            buf.at[pl.ds(bb * 56, _HIST)],
            out_hbm.at[b],
            sem.at[sem_slot],
        )

    def row_fill(bb, _c):
        for s, ts in _CHUNKS:
            mv = xv[bb, pl.ds(s, 16)]
            for t in ts:
                msp = mv.at[jnp.full((16,), t, jnp.int32)].get(
                    mode="promise_in_bounds")
                row = bb * 56 + (s + t)
                for c in range(4):
                    buf[row, pl.ds(c * 16, 16)] = jnp.where(
                        msp == 1, q1[c], q0[c])
        return _c

    def step(it, _):
        b0 = wid * _BPW + it * _CB

        @pl.when(it >= 1)
        def _drain_prev():
            for bb in range(_CB):
                out_copy(bb, b0 - _CB + bb, bb // _HB).wait()

        pltpu.sync_copy(x_hbm.at[pl.ds(b0, _CB)], xv)
        lax.fori_loop(0, _HB, row_fill, 0)
        for bb in range(_HB):
            out_copy(bb, b0 + bb, 0).start()
        lax.fori_loop(_HB, _CB, row_fill, 0)
        for bb in range(_HB, _CB):
            out_copy(bb, b0 + bb, 1).start()
        return 0

    lax.fori_loop(0, _ITERS, step, 0)
    bL = wid * _BPW + (_ITERS - 1) * _CB
    for bb in range(_CB):
        out_copy(bb, bL + bb, bb // _HB).wait()


def kernel(x, table):
    xi = x.astype(jnp.int32)
    tbl2 = jnp.pad(table[:2], ((0, 0), (0, 128 - _DIM)))
    mesh = plsc.VectorSubcoreMesh(core_axis_name="c", subcore_axis_name="s")
    k = pl.kernel(
        _sc_body,
        out_type=jax.ShapeDtypeStruct((_BATCH, _HIST, _DIM), jnp.float32),
        mesh=mesh,
        scratch_types=[
            pltpu.VMEM((_CB, _HIST), jnp.int32),
            pltpu.VMEM((2, 128), jnp.float32),
            pltpu.VMEM((_CB * 56, _DIM), jnp.float32),
            pltpu.SemaphoreType.DMA((2,)),
        ],
        compiler_params=pltpu.CompilerParams(
            needs_layout_passes=False, use_tc_tiling_on_sc=True),
    )
    return k(xi, tbl2)
